# Initial kernel scaffold; baseline (speedup 1.0000x reference)
import functools

import jax
import jax.numpy as jnp
from jax import lax
from jax.experimental import pallas as pl
from jax.experimental.pallas import tpu as pltpu
from jax.experimental.pallas import tpu_sc as plsc

N = 10000
E = 320000
IN = 128
HID = 16
HEADS = 8
OUT = 40

NPAD = 10240
PADROW = N
NCORES = 2
NSUB = 16
NTILES = NCORES * NSUB
B = 128
G = 81
C = B * G
ET = NTILES * C

D1 = 144
D2 = 48
DD = 16

_SC_MESH = dict(core_axis_name="c", subcore_axis_name="s", num_cores=NCORES,
                num_subcores=NSUB)


def _tca_body(x_ref, w1_ref, msrc_ref, mdst_ref, ts1_ref, td1_ref):
    h = jnp.dot(x_ref[...], w1_ref[...], preferred_element_type=jnp.float32)
    a_src = jnp.dot(h, msrc_ref[...], preferred_element_type=jnp.float32)
    ts1_ref[...] = jnp.concatenate([h, a_src], axis=1)
    td1_ref[...] = jnp.dot(h, mdst_ref[...], preferred_element_type=jnp.float32)


def _tcb_body(acc_ref, rep_ref, b1_ref, w2_ref, p2_ref, q2_ref,
              ts2_ref, td2_ref):
    num = acc_ref[0, :, :IN] + acc_ref[1, :, :IN]
    den = acc_ref[0, :, IN:IN + HEADS] + acc_ref[1, :, IN:IN + HEADS]
    denx = jnp.dot(den, rep_ref[...], preferred_element_type=jnp.float32)
    out1 = num / (denx + 1e-16) + b1_ref[...]
    h1f = jnp.where(out1 > 0, out1, jnp.expm1(out1))
    h2 = jnp.dot(h1f, w2_ref[...], preferred_element_type=jnp.float32)
    ts2_ref[...] = jnp.dot(h2, p2_ref[...], preferred_element_type=jnp.float32)
    td2_ref[...] = jnp.dot(h2, q2_ref[...], preferred_element_type=jnp.float32)


def _tcc_body(acc_ref, b2_ref, out_ref):
    num = acc_ref[0, :, :OUT] + acc_ref[1, :, :OUT]
    den = acc_ref[0, :, OUT:OUT + 1] + acc_ref[1, :, OUT:OUT + 1]
    logits = num / (den + 1e-16) + b2_ref[...]
    m = jnp.max(logits, axis=1, keepdims=True)
    s = logits - m
    lse = jnp.log(jnp.sum(jnp.exp(s), axis=1, keepdims=True))
    out_ref[...] = s - lse


def _sc_edge_body(ts_hbm, td_hbm, src_hbm, dst_hbm, out_hbm,
                  rows_v, adv_v, sidx_v, didx_v, wbuf_v, tmp_v, acc_sh,
                  sem, sem2, *, dcols, compute_chunk):
    cid = lax.axis_index("c")
    sid = lax.axis_index("s")
    tile = cid * NSUB + sid
    base = tile * C
    rows_per_tile = NPAD // NSUB
    wo_chunks = rows_per_tile // 64

    z16 = jnp.zeros((16,), jnp.float32)

    @pl.loop(0, 64)
    def _zero_tmp(r):
        for c0 in range(dcols // 16):
            tmp_v[r, pl.ds(c0 * 16, 16)] = z16

    @pl.loop(0, wo_chunks)
    def _zero_acc(j):
        pltpu.sync_copy(tmp_v,
                        acc_sh.at[pl.ds(sid * rows_per_tile + j * 64, 64)])

    plsc.subcore_barrier()

    @pl.loop(0, G)
    def _chunk(g):
        goff = base + g * B
        pltpu.sync_copy(src_hbm.at[pl.ds(goff, B)], sidx_v)
        pltpu.sync_copy(dst_hbm.at[pl.ds(goff, B)], didx_v)
        cp1 = pltpu.async_copy(ts_hbm.at[sidx_v], rows_v, sem)
        cp2 = pltpu.async_copy(td_hbm.at[didx_v], adv_v, sem2)
        cp1.wait()
        cp2.wait()
        compute_chunk(rows_v, adv_v, wbuf_v)
        pltpu.sync_copy(rows_v, acc_sh.at[didx_v], add=True)

    plsc.subcore_barrier()

    @pl.loop(0, wo_chunks)
    def _writeout(j):
        r0 = sid * rows_per_tile + j * 64
        pltpu.sync_copy(acc_sh.at[pl.ds(r0, 64)], tmp_v)
        pltpu.sync_copy(tmp_v, out_hbm.at[cid, pl.ds(r0, 64)])


def _compute_chunk_l1(rows_v, adv_v, wbuf_v):
    @pl.loop(0, B)
    def _edge(e):
        as16 = rows_v[e, pl.ds(IN, 16)]
        ad16 = adv_v[e, pl.ds(0, 16)]
        al = as16 + ad16
        al = jnp.where(al > 0, al, 0.2 * al)
        w16 = jnp.exp(al)
        rows_v[e, pl.ds(IN, 16)] = w16
        for h in range(HEADS):
            wv = rows_v[e, IN + h]
            rows_v[e, pl.ds(h * 16, 16)] = rows_v[e, pl.ds(h * 16, 16)] * wv


def _compute_chunk_l2(rows_v, adv_v, wbuf_v):
    lane = lax.iota(jnp.int32, 16)
    is8 = lane == 8

    @pl.loop(0, B)
    def _edge(e):
        r2 = rows_v[e, pl.ds(32, 16)]
        ad16 = adv_v[e, pl.ds(0, 16)]
        s = r2 + ad16
        al = jnp.where(s > 0, s, 0.2 * s)
        w16 = jnp.exp(al)
        wbuf_v[pl.ds(0, 16)] = w16
        wv = wbuf_v[8]
        t2 = jnp.where(is8, w16, r2 * wv)
        rows_v[e, pl.ds(0, 16)] = rows_v[e, pl.ds(0, 16)] * wv
        rows_v[e, pl.ds(16, 16)] = rows_v[e, pl.ds(16, 16)] * wv
        rows_v[e, pl.ds(32, 16)] = t2


def _make_sc_kernel(dcols, compute_chunk, name):
    return pl.kernel(
        functools.partial(_sc_edge_body, dcols=dcols,
                          compute_chunk=compute_chunk),
        out_type=jax.ShapeDtypeStruct((NCORES, NPAD, dcols), jnp.float32),
        mesh=plsc.VectorSubcoreMesh(**_SC_MESH),
        scratch_types=[
            pltpu.VMEM((B, dcols), jnp.float32),
            pltpu.VMEM((B, DD), jnp.float32),
            pltpu.VMEM((B,), jnp.int32),
            pltpu.VMEM((B,), jnp.int32),
            pltpu.VMEM((16,), jnp.float32),
            pltpu.VMEM((64, dcols), jnp.float32),
            pltpu.VMEM_SHARED((NPAD, dcols), jnp.float32),
            pltpu.SemaphoreType.DMA,
            pltpu.SemaphoreType.DMA,
        ],
        name=name,
    )


def kernel(x, edge_index, W1, att_src1, att_dst1, b1, W2, att_src2, att_dst2,
           b2):
    f32 = jnp.float32
    loop = jnp.arange(N, dtype=jnp.int32)
    padv = jnp.full((ET - E - N,), PADROW, dtype=jnp.int32)
    src = jnp.concatenate([edge_index[0].astype(jnp.int32), loop, padv])
    dst = jnp.concatenate([edge_index[1].astype(jnp.int32), loop, padv])

    x_pad = jnp.zeros((NPAD, IN), f32).at[:N].set(x)

    eye8 = jnp.eye(HEADS, 16, dtype=f32)
    msrc = jnp.einsum("hd,hc->hdc", att_src1, eye8).reshape(IN, 16)
    mdst = jnp.einsum("hd,hc->hdc", att_dst1, eye8).reshape(IN, 16)
    rep = jnp.kron(jnp.eye(HEADS, dtype=f32), jnp.ones((1, HID), f32))
    p2 = jnp.concatenate(
        [jnp.eye(OUT, dtype=f32), att_src2.T, jnp.zeros((OUT, 7), f32)],
        axis=1)
    q2 = jnp.concatenate(
        [jnp.zeros((OUT, 8), f32), att_dst2.T, jnp.zeros((OUT, 7), f32)],
        axis=1)
    b1r = b1.reshape(1, IN)
    b2r = b2.reshape(1, OUT)

    RB = 640
    grid = NPAD // RB

    ts1, td1 = pl.pallas_call(
        _tca_body,
        grid=(grid,),
        in_specs=[
            pl.BlockSpec((RB, IN), lambda i: (i, 0)),
            pl.BlockSpec((IN, IN), lambda i: (0, 0)),
            pl.BlockSpec((IN, 16), lambda i: (0, 0)),
            pl.BlockSpec((IN, 16), lambda i: (0, 0)),
        ],
        out_specs=[
            pl.BlockSpec((RB, D1), lambda i: (i, 0)),
            pl.BlockSpec((RB, DD), lambda i: (i, 0)),
        ],
        out_shape=[
            jax.ShapeDtypeStruct((NPAD, D1), f32),
            jax.ShapeDtypeStruct((NPAD, DD), f32),
        ],
    )(x_pad, W1, msrc, mdst)

    acc1 = _make_sc_kernel(D1, _compute_chunk_l1, "sc_gat_l1")(
        ts1, td1, src, dst)

    ts2, td2 = pl.pallas_call(
        _tcb_body,
        grid=(grid,),
        in_specs=[
            pl.BlockSpec((NCORES, RB, D1), lambda i: (0, i, 0)),
            pl.BlockSpec((HEADS, IN), lambda i: (0, 0)),
            pl.BlockSpec((1, IN), lambda i: (0, 0)),
            pl.BlockSpec((IN, OUT), lambda i: (0, 0)),
            pl.BlockSpec((OUT, D2), lambda i: (0, 0)),
            pl.BlockSpec((OUT, DD), lambda i: (0, 0)),
        ],
        out_specs=[
            pl.BlockSpec((RB, D2), lambda i: (i, 0)),
            pl.BlockSpec((RB, DD), lambda i: (i, 0)),
        ],
        out_shape=[
            jax.ShapeDtypeStruct((NPAD, D2), f32),
            jax.ShapeDtypeStruct((NPAD, DD), f32),
        ],
    )(acc1, rep, b1r, W2, p2, q2)

    acc2 = _make_sc_kernel(D2, _compute_chunk_l2, "sc_gat_l2")(
        ts2, td2, src, dst)

    out = pl.pallas_call(
        _tcc_body,
        grid=(grid,),
        in_specs=[
            pl.BlockSpec((NCORES, RB, D2), lambda i: (0, i, 0)),
            pl.BlockSpec((1, OUT), lambda i: (0, 0)),
        ],
        out_specs=pl.BlockSpec((RB, OUT), lambda i: (i, 0)),
        out_shape=jax.ShapeDtypeStruct((NPAD, OUT), f32),
    )(acc2, b2r)

    return out[:N]


# speedup vs baseline: 49.3936x; 49.3936x over previous
import functools

import jax
import jax.numpy as jnp
from jax import lax
from jax.experimental import pallas as pl
from jax.experimental.pallas import tpu as pltpu
from jax.experimental.pallas import tpu_sc as plsc

N = 10000
E = 320000
IN = 128
HID = 16
HEADS = 8
OUT = 40

NPAD = 10240
PADROW = N
NCORES = 2
NSUB = 16
NTILES = NCORES * NSUB
B = 128
G = 81
C = B * G
ET = NTILES * C

D1 = 144
D2 = 48
DD = 16

_SC_MESH = dict(core_axis_name="c", subcore_axis_name="s", num_cores=NCORES,
                num_subcores=NSUB)


def _tca_body(x_ref, w1_ref, msrc_ref, mdst_ref, ts1_ref, td1_ref):
    h = jnp.dot(x_ref[...], w1_ref[...], preferred_element_type=jnp.float32)
    a_src = jnp.dot(h, msrc_ref[...], preferred_element_type=jnp.float32)
    ts1_ref[...] = jnp.concatenate([h, a_src], axis=1)
    td1_ref[...] = jnp.dot(h, mdst_ref[...], preferred_element_type=jnp.float32)


def _tcb_body(acc_ref, rep_ref, b1_ref, w2_ref, p2_ref, q2_ref,
              ts2_ref, td2_ref):
    num = acc_ref[0, :, :IN] + acc_ref[1, :, :IN]
    den = acc_ref[0, :, IN:IN + HEADS] + acc_ref[1, :, IN:IN + HEADS]
    denx = jnp.dot(den, rep_ref[...], preferred_element_type=jnp.float32)
    out1 = num / (denx + 1e-16) + b1_ref[...]
    h1f = jnp.where(out1 > 0, out1, jnp.exp(out1) - 1.0)
    h2 = jnp.dot(h1f, w2_ref[...], preferred_element_type=jnp.float32)
    ts2_ref[...] = jnp.dot(h2, p2_ref[...], preferred_element_type=jnp.float32)
    td2_ref[...] = jnp.dot(h2, q2_ref[...], preferred_element_type=jnp.float32)


def _tcc_body(acc_ref, b2_ref, out_ref):
    num = acc_ref[0, :, :OUT] + acc_ref[1, :, :OUT]
    den = acc_ref[0, :, OUT:OUT + 1] + acc_ref[1, :, OUT:OUT + 1]
    logits = num / (den + 1e-16) + b2_ref[...]
    m = jnp.max(logits, axis=1, keepdims=True)
    s = logits - m
    lse = jnp.log(jnp.sum(jnp.exp(s), axis=1, keepdims=True))
    out_ref[...] = s - lse


def _sc_edge_body(ts_hbm, td_hbm, src_hbm, dst_hbm, out_hbm,
                  rows_v, adv_v, sidx_v, didx_v, wbuf_v, tmp_v, acc_sh,
                  sem, sem2, *, dcols, compute_chunk):
    cid = lax.axis_index("c")
    sid = lax.axis_index("s")
    tile = cid * NSUB + sid
    base = tile * C
    rows_per_tile = NPAD // NSUB
    wo_chunks = rows_per_tile // 64

    z16 = jnp.zeros((16,), jnp.float32)

    @pl.loop(0, 64)
    def _zero_tmp(r):
        for c0 in range(dcols // 16):
            tmp_v[r, pl.ds(c0 * 16, 16)] = z16

    @pl.loop(0, wo_chunks)
    def _zero_acc(j):
        pltpu.sync_copy(tmp_v,
                        acc_sh.at[pl.ds(sid * rows_per_tile + j * 64, 64)])

    plsc.subcore_barrier()

    @pl.loop(0, G)
    def _chunk(g):
        goff = base + g * B
        pltpu.sync_copy(src_hbm.at[pl.ds(goff, B)], sidx_v)
        pltpu.sync_copy(dst_hbm.at[pl.ds(goff, B)], didx_v)
        cp1 = pltpu.async_copy(ts_hbm.at[sidx_v], rows_v, sem)
        cp2 = pltpu.async_copy(td_hbm.at[didx_v], adv_v, sem2)
        cp1.wait()
        cp2.wait()
        compute_chunk(rows_v, adv_v, wbuf_v)
        pltpu.sync_copy(rows_v, acc_sh.at[didx_v], add=True)

    plsc.subcore_barrier()

    @pl.loop(0, wo_chunks)
    def _writeout(j):
        r0 = sid * rows_per_tile + j * 64
        pltpu.sync_copy(acc_sh.at[pl.ds(r0, 64)], tmp_v)
        pltpu.sync_copy(tmp_v, out_hbm.at[cid, pl.ds(r0, 64)])


def _compute_chunk_l1(rows_v, adv_v, wbuf_v):
    @pl.loop(0, B)
    def _edge(e):
        as16 = rows_v[e, pl.ds(IN, 16)]
        ad16 = adv_v[e, pl.ds(0, 16)]
        al = as16 + ad16
        al = jnp.where(al > 0, al, 0.2 * al)
        w16 = jnp.exp(al)
        rows_v[e, pl.ds(IN, 16)] = w16
        for h in range(HEADS):
            wv = w16[h]
            rows_v[e, pl.ds(h * 16, 16)] = rows_v[e, pl.ds(h * 16, 16)] * wv


def _compute_chunk_l2(rows_v, adv_v, wbuf_v):
    lane = lax.iota(jnp.int32, 16)
    is8 = lane == 8

    @pl.loop(0, B)
    def _edge(e):
        r2 = rows_v[e, pl.ds(32, 16)]
        ad16 = adv_v[e, pl.ds(0, 16)]
        s = r2 + ad16
        al = jnp.where(s > 0, s, 0.2 * s)
        w16 = jnp.exp(al)
        wv = w16[8]
        t2 = jnp.where(is8, w16, r2 * wv)
        rows_v[e, pl.ds(0, 16)] = rows_v[e, pl.ds(0, 16)] * wv
        rows_v[e, pl.ds(16, 16)] = rows_v[e, pl.ds(16, 16)] * wv
        rows_v[e, pl.ds(32, 16)] = t2


def _make_sc_kernel(dcols, compute_chunk, name):
    return pl.kernel(
        functools.partial(_sc_edge_body, dcols=dcols,
                          compute_chunk=compute_chunk),
        out_type=jax.ShapeDtypeStruct((NCORES, NPAD, dcols), jnp.float32),
        mesh=plsc.VectorSubcoreMesh(**_SC_MESH),
        scratch_types=[
            pltpu.VMEM((B, dcols), jnp.float32),
            pltpu.VMEM((B, DD), jnp.float32),
            pltpu.VMEM((B,), jnp.int32),
            pltpu.VMEM((B,), jnp.int32),
            pltpu.VMEM((16,), jnp.float32),
            pltpu.VMEM((64, dcols), jnp.float32),
            pltpu.VMEM_SHARED((NPAD, dcols), jnp.float32),
            pltpu.SemaphoreType.DMA,
            pltpu.SemaphoreType.DMA,
        ],
        compiler_params=pltpu.CompilerParams(use_tc_tiling_on_sc=False),
        name=name,
    )


def kernel(x, edge_index, W1, att_src1, att_dst1, b1, W2, att_src2, att_dst2,
           b2):
    f32 = jnp.float32
    loop = jnp.arange(N, dtype=jnp.int32)
    padv = jnp.full((ET - E - N,), PADROW, dtype=jnp.int32)
    src = jnp.concatenate([edge_index[0].astype(jnp.int32), loop, padv])
    dst = jnp.concatenate([edge_index[1].astype(jnp.int32), loop, padv])

    x_pad = jnp.zeros((NPAD, IN), f32).at[:N].set(x)

    eye8 = jnp.eye(HEADS, 16, dtype=f32)
    msrc = jnp.einsum("hd,hc->hdc", att_src1, eye8).reshape(IN, 16)
    mdst = jnp.einsum("hd,hc->hdc", att_dst1, eye8).reshape(IN, 16)
    rep = jnp.kron(jnp.eye(HEADS, dtype=f32), jnp.ones((1, HID), f32))
    p2 = jnp.concatenate(
        [jnp.eye(OUT, dtype=f32), att_src2.T, jnp.zeros((OUT, 7), f32)],
        axis=1)
    q2 = jnp.concatenate(
        [jnp.zeros((OUT, 8), f32), att_dst2.T, jnp.zeros((OUT, 7), f32)],
        axis=1)
    b1r = b1.reshape(1, IN)
    b2r = b2.reshape(1, OUT)

    RB = 640
    grid = NPAD // RB

    ts1, td1 = pl.pallas_call(
        _tca_body,
        grid=(grid,),
        in_specs=[
            pl.BlockSpec((RB, IN), lambda i: (i, 0)),
            pl.BlockSpec((IN, IN), lambda i: (0, 0)),
            pl.BlockSpec((IN, 16), lambda i: (0, 0)),
            pl.BlockSpec((IN, 16), lambda i: (0, 0)),
        ],
        out_specs=[
            pl.BlockSpec((RB, D1), lambda i: (i, 0)),
            pl.BlockSpec((RB, DD), lambda i: (i, 0)),
        ],
        out_shape=[
            jax.ShapeDtypeStruct((NPAD, D1), f32),
            jax.ShapeDtypeStruct((NPAD, DD), f32),
        ],
    )(x_pad, W1, msrc, mdst)

    acc1 = _make_sc_kernel(D1, _compute_chunk_l1, "sc_gat_l1")(
        ts1, td1, src, dst)

    ts2, td2 = pl.pallas_call(
        _tcb_body,
        grid=(grid,),
        in_specs=[
            pl.BlockSpec((NCORES, RB, D1), lambda i: (0, i, 0)),
            pl.BlockSpec((HEADS, IN), lambda i: (0, 0)),
            pl.BlockSpec((1, IN), lambda i: (0, 0)),
            pl.BlockSpec((IN, OUT), lambda i: (0, 0)),
            pl.BlockSpec((OUT, D2), lambda i: (0, 0)),
            pl.BlockSpec((OUT, DD), lambda i: (0, 0)),
        ],
        out_specs=[
            pl.BlockSpec((RB, D2), lambda i: (i, 0)),
            pl.BlockSpec((RB, DD), lambda i: (i, 0)),
        ],
        out_shape=[
            jax.ShapeDtypeStruct((NPAD, D2), f32),
            jax.ShapeDtypeStruct((NPAD, DD), f32),
        ],
    )(acc1, rep, b1r, W2, p2, q2)

    acc2 = _make_sc_kernel(D2, _compute_chunk_l2, "sc_gat_l2")(
        ts2, td2, src, dst)

    out = pl.pallas_call(
        _tcc_body,
        grid=(grid,),
        in_specs=[
            pl.BlockSpec((NCORES, RB, D2), lambda i: (0, i, 0)),
            pl.BlockSpec((1, OUT), lambda i: (0, 0)),
        ],
        out_specs=pl.BlockSpec((RB, OUT), lambda i: (i, 0)),
        out_shape=jax.ShapeDtypeStruct((NPAD, OUT), f32),
    )(acc2, b2r)

    return out[:N]


# speedup vs baseline: 78.3569x; 1.5864x over previous
import functools

import jax
import jax.numpy as jnp
from jax import lax
from jax.experimental import pallas as pl
from jax.experimental.pallas import tpu as pltpu
from jax.experimental.pallas import tpu_sc as plsc

N = 10000
E = 320000
IN = 128
HID = 16
HEADS = 8
OUT = 40

NPAD = 10240
PADROW = N
NCORES = 2
NSUB = 16
NTILES = NCORES * NSUB
B = 64
G = 162
C = B * G
ET = NTILES * C

D1 = 144
D2 = 48
DD = 16

_SC_MESH = dict(core_axis_name="c", subcore_axis_name="s", num_cores=NCORES,
                num_subcores=NSUB)


def _tca_body(x_ref, w1_ref, msrc_ref, mdst_ref, ts1_ref, td1_ref):
    h = jnp.dot(x_ref[...], w1_ref[...], preferred_element_type=jnp.float32)
    a_src = jnp.dot(h, msrc_ref[...], preferred_element_type=jnp.float32)
    ts1_ref[...] = jnp.concatenate([h, a_src], axis=1)
    td1_ref[...] = jnp.dot(h, mdst_ref[...], preferred_element_type=jnp.float32)


def _tcb_body(acc_ref, rep_ref, b1_ref, w2_ref, p2_ref, q2_ref,
              ts2_ref, td2_ref):
    num = acc_ref[0, :, :IN] + acc_ref[1, :, :IN]
    den = acc_ref[0, :, IN:IN + HEADS] + acc_ref[1, :, IN:IN + HEADS]
    denx = jnp.dot(den, rep_ref[...], preferred_element_type=jnp.float32)
    out1 = num / (denx + 1e-16) + b1_ref[...]
    h1f = jnp.where(out1 > 0, out1, jnp.exp(out1) - 1.0)
    h2 = jnp.dot(h1f, w2_ref[...], preferred_element_type=jnp.float32)
    ts2_ref[...] = jnp.dot(h2, p2_ref[...], preferred_element_type=jnp.float32)
    td2_ref[...] = jnp.dot(h2, q2_ref[...], preferred_element_type=jnp.float32)


def _tcc_body(acc_ref, b2_ref, out_ref):
    num = acc_ref[0, :, :OUT] + acc_ref[1, :, :OUT]
    den = acc_ref[0, :, OUT:OUT + 1] + acc_ref[1, :, OUT:OUT + 1]
    logits = num / (den + 1e-16) + b2_ref[...]
    m = jnp.max(logits, axis=1, keepdims=True)
    s = logits - m
    lse = jnp.log(jnp.sum(jnp.exp(s), axis=1, keepdims=True))
    out_ref[...] = s - lse


def _sc_edge_body(ts_hbm, td_hbm, src_hbm, dst_hbm, out_hbm,
                  rows0, rows1, rows2, adv0, adv1, adv2,
                  sidx_v, didx_v, acc_sh,
                  gs0, gs1, gs2, ss0, ss1, ss2,
                  is0, is1, is2, is3, is4, is5,
                  *, dcols, compute_chunk):
    cid = lax.axis_index("c")
    sid = lax.axis_index("s")
    tile = cid * NSUB + sid
    rpt = NPAD // NSUB

    rows_b = (rows0, rows1, rows2)
    adv_b = (adv0, adv1, adv2)
    gsem = (gs0, gs1, gs2)
    ssem = (ss0, ss1, ss2)
    isem = (is0, is1, is2, is3, is4, is5)

    def idx_issue(c, s6):
        pltpu.async_copy(src_hbm.at[tile * G + c], sidx_v.at[s6], isem[s6])
        pltpu.async_copy(dst_hbm.at[tile * G + c], didx_v.at[s6], isem[s6])

    def idx_wait(c, s6):
        pltpu.make_async_copy(src_hbm.at[tile * G + c], sidx_v.at[s6],
                              isem[s6]).wait()
        pltpu.make_async_copy(dst_hbm.at[tile * G + c], didx_v.at[s6],
                              isem[s6]).wait()

    def g_issue(s6, b3):
        pltpu.async_copy(ts_hbm.at[sidx_v.at[s6]], rows_b[b3], gsem[b3])
        pltpu.async_copy(td_hbm.at[didx_v.at[s6]], adv_b[b3], gsem[b3])

    def g_wait(s6, b3):
        pltpu.make_async_copy(ts_hbm.at[sidx_v.at[s6]], rows_b[b3],
                              gsem[b3]).wait()
        pltpu.make_async_copy(td_hbm.at[didx_v.at[s6]], adv_b[b3],
                              gsem[b3]).wait()

    def s_issue(s6, b3):
        pltpu.async_copy(rows_b[b3], acc_sh.at[didx_v.at[s6]], ssem[b3],
                         add=True)

    def s_wait(s6, b3):
        pltpu.make_async_copy(rows_b[b3], acc_sh.at[didx_v.at[s6]],
                              ssem[b3]).wait()

    def step(g, u, first=False, no_next=False, no_idx3=False):
        b = u % 3
        g_wait(u, b)
        if not first:
            s_wait((u + 4) % 6, (u + 1) % 3)
        if not no_next:
            idx_wait(g + 1, (u + 1) % 6)
            g_issue((u + 1) % 6, (u + 1) % 3)
        if not no_idx3:
            idx_issue(g + 3, (u + 3) % 6)
        compute_chunk(rows_b[b], adv_b[b])
        s_issue(u, b)

    for c in range(3):
        idx_issue(c, c)
    idx_wait(0, 0)
    g_issue(0, 0)

    z16 = jnp.zeros((16,), jnp.float32)

    @pl.loop(0, B)
    def _zero_rows2(r):
        for c0 in range(dcols // 16):
            rows2[r, pl.ds(c0 * 16, 16)] = z16

    @pl.loop(0, rpt // B)
    def _zero_acc(j):
        pltpu.sync_copy(rows2, acc_sh.at[pl.ds(sid * rpt + j * B, B)])

    plsc.subcore_barrier()

    for u in range(6):
        step(u, u, first=(u < 2))

    @pl.loop(1, G // 6 - 1)
    def _six(i):
        g0 = i * 6
        for u in range(6):
            step(g0 + u, u)

    for u in range(6):
        step(G - 6 + u, u, no_next=(u >= 5), no_idx3=(u >= 3))

    s_wait(4, 1)
    s_wait(5, 2)

    plsc.subcore_barrier()

    pltpu.sync_copy(acc_sh.at[pl.ds(sid * rpt, rpt)],
                    out_hbm.at[cid, pl.ds(sid * rpt, rpt)])


def _compute_chunk_l1(rows_v, adv_v):
    @pl.loop(0, B, unroll=4)
    def _edge(e):
        as16 = rows_v[e, pl.ds(IN, 16)]
        ad16 = adv_v[e, pl.ds(0, 16)]
        al = as16 + ad16
        al = jnp.where(al > 0, al, 0.2 * al)
        w16 = jnp.exp(al)
        rows_v[e, pl.ds(IN, 16)] = w16
        for h in range(HEADS):
            wv = w16[h]
            rows_v[e, pl.ds(h * 16, 16)] = rows_v[e, pl.ds(h * 16, 16)] * wv


def _compute_chunk_l2(rows_v, adv_v):
    lane = lax.iota(jnp.int32, 16)
    is8 = lane == 8

    @pl.loop(0, B, unroll=4)
    def _edge(e):
        r2 = rows_v[e, pl.ds(32, 16)]
        ad16 = adv_v[e, pl.ds(0, 16)]
        s = r2 + ad16
        al = jnp.where(s > 0, s, 0.2 * s)
        w16 = jnp.exp(al)
        wv = w16[8]
        rows_v[e, pl.ds(0, 16)] = rows_v[e, pl.ds(0, 16)] * wv
        rows_v[e, pl.ds(16, 16)] = rows_v[e, pl.ds(16, 16)] * wv
        rows_v[e, pl.ds(32, 16)] = jnp.where(is8, w16, r2 * wv)


def _make_sc_kernel(dcols, compute_chunk, name):
    return pl.kernel(
        functools.partial(_sc_edge_body, dcols=dcols,
                          compute_chunk=compute_chunk),
        out_type=jax.ShapeDtypeStruct((NCORES, NPAD, dcols), jnp.float32),
        mesh=plsc.VectorSubcoreMesh(**_SC_MESH),
        scratch_types=[
            pltpu.VMEM((B, dcols), jnp.float32),
            pltpu.VMEM((B, dcols), jnp.float32),
            pltpu.VMEM((B, dcols), jnp.float32),
            pltpu.VMEM((B, DD), jnp.float32),
            pltpu.VMEM((B, DD), jnp.float32),
            pltpu.VMEM((B, DD), jnp.float32),
            pltpu.VMEM((6, B), jnp.int32),
            pltpu.VMEM((6, B), jnp.int32),
            pltpu.VMEM_SHARED((NPAD, dcols), jnp.float32),
            pltpu.SemaphoreType.DMA,
            pltpu.SemaphoreType.DMA,
            pltpu.SemaphoreType.DMA,
            pltpu.SemaphoreType.DMA,
            pltpu.SemaphoreType.DMA,
            pltpu.SemaphoreType.DMA,
            pltpu.SemaphoreType.DMA,
            pltpu.SemaphoreType.DMA,
            pltpu.SemaphoreType.DMA,
            pltpu.SemaphoreType.DMA,
            pltpu.SemaphoreType.DMA,
            pltpu.SemaphoreType.DMA,
        ],
        compiler_params=pltpu.CompilerParams(use_tc_tiling_on_sc=False),
        name=name,
    )


def kernel(x, edge_index, W1, att_src1, att_dst1, b1, W2, att_src2, att_dst2,
           b2):
    f32 = jnp.float32
    loop = jnp.arange(N, dtype=jnp.int32)
    padv = jnp.full((ET - E - N,), PADROW, dtype=jnp.int32)
    src = jnp.concatenate([edge_index[0].astype(jnp.int32), loop, padv])
    dst = jnp.concatenate([edge_index[1].astype(jnp.int32), loop, padv])
    src = src.reshape(NTILES * G, B)
    dst = dst.reshape(NTILES * G, B)

    x_pad = jnp.zeros((NPAD, IN), f32).at[:N].set(x)

    eye8 = jnp.eye(HEADS, 16, dtype=f32)
    msrc = jnp.einsum("hd,hc->hdc", att_src1, eye8).reshape(IN, 16)
    mdst = jnp.einsum("hd,hc->hdc", att_dst1, eye8).reshape(IN, 16)
    rep = jnp.kron(jnp.eye(HEADS, dtype=f32), jnp.ones((1, HID), f32))
    p2 = jnp.concatenate(
        [jnp.eye(OUT, dtype=f32), att_src2.T, jnp.zeros((OUT, 7), f32)],
        axis=1)
    q2 = jnp.concatenate(
        [jnp.zeros((OUT, 8), f32), att_dst2.T, jnp.zeros((OUT, 7), f32)],
        axis=1)
    b1r = b1.reshape(1, IN)
    b2r = b2.reshape(1, OUT)

    RB = 640
    grid = NPAD // RB

    ts1, td1 = pl.pallas_call(
        _tca_body,
        grid=(grid,),
        in_specs=[
            pl.BlockSpec((RB, IN), lambda i: (i, 0)),
            pl.BlockSpec((IN, IN), lambda i: (0, 0)),
            pl.BlockSpec((IN, 16), lambda i: (0, 0)),
            pl.BlockSpec((IN, 16), lambda i: (0, 0)),
        ],
        out_specs=[
            pl.BlockSpec((RB, D1), lambda i: (i, 0)),
            pl.BlockSpec((RB, DD), lambda i: (i, 0)),
        ],
        out_shape=[
            jax.ShapeDtypeStruct((NPAD, D1), f32),
            jax.ShapeDtypeStruct((NPAD, DD), f32),
        ],
    )(x_pad, W1, msrc, mdst)

    acc1 = _make_sc_kernel(D1, _compute_chunk_l1, "sc_gat_l1")(
        ts1, td1, src, dst)

    ts2, td2 = pl.pallas_call(
        _tcb_body,
        grid=(grid,),
        in_specs=[
            pl.BlockSpec((NCORES, RB, D1), lambda i: (0, i, 0)),
            pl.BlockSpec((HEADS, IN), lambda i: (0, 0)),
            pl.BlockSpec((1, IN), lambda i: (0, 0)),
            pl.BlockSpec((IN, OUT), lambda i: (0, 0)),
            pl.BlockSpec((OUT, D2), lambda i: (0, 0)),
            pl.BlockSpec((OUT, DD), lambda i: (0, 0)),
        ],
        out_specs=[
            pl.BlockSpec((RB, D2), lambda i: (i, 0)),
            pl.BlockSpec((RB, DD), lambda i: (i, 0)),
        ],
        out_shape=[
            jax.ShapeDtypeStruct((NPAD, D2), f32),
            jax.ShapeDtypeStruct((NPAD, DD), f32),
        ],
    )(acc1, rep, b1r, W2, p2, q2)

    acc2 = _make_sc_kernel(D2, _compute_chunk_l2, "sc_gat_l2")(
        ts2, td2, src, dst)

    out = pl.pallas_call(
        _tcc_body,
        grid=(grid,),
        in_specs=[
            pl.BlockSpec((NCORES, RB, D2), lambda i: (0, i, 0)),
            pl.BlockSpec((1, OUT), lambda i: (0, 0)),
        ],
        out_specs=pl.BlockSpec((RB, OUT), lambda i: (i, 0)),
        out_shape=jax.ShapeDtypeStruct((NPAD, OUT), f32),
    )(acc2, b2r)

    return out[:N]


# speedup vs baseline: 92.6408x; 1.1823x over previous
import functools

import jax
import jax.numpy as jnp
from jax import lax
from jax.experimental import pallas as pl
from jax.experimental.pallas import tpu as pltpu
from jax.experimental.pallas import tpu_sc as plsc

N = 10000
E = 320000
IN = 128
HID = 16
HEADS = 8
OUT = 40

NPAD = 10240
PADROW = N
NCORES = 2
NSUB = 16
NTILES = NCORES * NSUB
B1 = 64
G1 = 168
B2 = 128
G2 = 84
C = B1 * G1
ET = NTILES * C
NPADROWS = 240

D1 = 144
D2 = 48
DD = 16

_SC_MESH = dict(core_axis_name="c", subcore_axis_name="s", num_cores=NCORES,
                num_subcores=NSUB)


def _tca_body(x_ref, w1_ref, msrc_ref, mdst_ref, ts1_ref, td1_ref):
    h = jnp.dot(x_ref[...], w1_ref[...], preferred_element_type=jnp.float32)
    a_src = jnp.dot(h, msrc_ref[...], preferred_element_type=jnp.float32)
    ts1_ref[...] = jnp.concatenate([h, a_src], axis=1)
    td1_ref[...] = jnp.dot(h, mdst_ref[...], preferred_element_type=jnp.float32)


def _tcb_body(acc_ref, rep_ref, b1_ref, w2_ref, p2_ref, q2_ref,
              ts2_ref, td2_ref):
    num = acc_ref[0, :, :IN] + acc_ref[1, :, :IN]
    den = acc_ref[0, :, IN:IN + HEADS] + acc_ref[1, :, IN:IN + HEADS]
    denx = jnp.dot(den, rep_ref[...], preferred_element_type=jnp.float32)
    out1 = num / (denx + 1e-16) + b1_ref[...]
    h1f = jnp.where(out1 > 0, out1, jnp.exp(out1) - 1.0)
    h2 = jnp.dot(h1f, w2_ref[...], preferred_element_type=jnp.float32)
    ts2_ref[...] = jnp.dot(h2, p2_ref[...], preferred_element_type=jnp.float32)
    td2_ref[...] = jnp.dot(h2, q2_ref[...], preferred_element_type=jnp.float32)


def _tcc_body(acc_ref, b2_ref, out_ref):
    num = acc_ref[0, :, :OUT] + acc_ref[1, :, :OUT]
    den = acc_ref[0, :, OUT:OUT + 1] + acc_ref[1, :, OUT:OUT + 1]
    logits = num / (den + 1e-16) + b2_ref[...]
    m = jnp.max(logits, axis=1, keepdims=True)
    s = logits - m
    lse = jnp.log(jnp.sum(jnp.exp(s), axis=1, keepdims=True))
    out_ref[...] = s - lse


def _sc_edge_body(ts_hbm, td_hbm, src_hbm, dst_hbm, out_hbm,
                  rows0, rows1, rows2, adv0, adv1, adv2,
                  sidx_v, didx_v, acc_sh,
                  gs0, gs1, gs2, ss0, ss1, ss2,
                  is0, is1, is2, is3, is4, is5,
                  *, dcols, bb, gg, compute_chunk):
    cid = lax.axis_index("c")
    sid = lax.axis_index("s")
    tile = cid * NSUB + sid
    rpt = NPAD // NSUB

    rows_b = (rows0, rows1, rows2)
    adv_b = (adv0, adv1, adv2)
    gsem = (gs0, gs1, gs2)
    ssem = (ss0, ss1, ss2)
    isem = (is0, is1, is2, is3, is4, is5)

    def idx_issue(c, s6):
        pltpu.async_copy(src_hbm.at[tile * gg + c], sidx_v.at[s6], isem[s6])
        pltpu.async_copy(dst_hbm.at[tile * gg + c], didx_v.at[s6], isem[s6])

    def idx_wait(c, s6):
        pltpu.make_async_copy(src_hbm.at[tile * gg + c], sidx_v.at[s6],
                              isem[s6]).wait()
        pltpu.make_async_copy(dst_hbm.at[tile * gg + c], didx_v.at[s6],
                              isem[s6]).wait()

    def g_issue(s6, b3):
        pltpu.async_copy(ts_hbm.at[sidx_v.at[s6]], rows_b[b3], gsem[b3])
        pltpu.async_copy(td_hbm.at[didx_v.at[s6]], adv_b[b3], gsem[b3])

    def g_wait(s6, b3):
        pltpu.make_async_copy(ts_hbm.at[sidx_v.at[s6]], rows_b[b3],
                              gsem[b3]).wait()
        pltpu.make_async_copy(td_hbm.at[didx_v.at[s6]], adv_b[b3],
                              gsem[b3]).wait()

    def s_issue(s6, b3):
        pltpu.async_copy(rows_b[b3], acc_sh.at[didx_v.at[s6]], ssem[b3],
                         add=True)

    def s_wait(s6, b3):
        pltpu.make_async_copy(rows_b[b3], acc_sh.at[didx_v.at[s6]],
                              ssem[b3]).wait()

    def step(g, u, first=False, no_next=False, no_idx3=False):
        b = u % 3
        g_wait(u, b)
        if not first:
            s_wait((u + 4) % 6, (u + 1) % 3)
        if not no_next:
            idx_wait(g + 1, (u + 1) % 6)
            g_issue((u + 1) % 6, (u + 1) % 3)
        if not no_idx3:
            idx_issue(g + 3, (u + 3) % 6)
        compute_chunk(rows_b[b], adv_b[b])
        s_issue(u, b)

    for c in range(3):
        idx_issue(c, c)
    idx_wait(0, 0)
    g_issue(0, 0)

    z16 = jnp.zeros((16,), jnp.float32)

    @pl.loop(0, bb)
    def _zero_rows2(r):
        for c0 in range(dcols // 16):
            rows2[r, pl.ds(c0 * 16, 16)] = z16

    @pl.loop(0, rpt // bb)
    def _zero_acc(j):
        pltpu.sync_copy(rows2, acc_sh.at[pl.ds(sid * rpt + j * bb, bb)])

    plsc.subcore_barrier()

    for u in range(6):
        step(u, u, first=(u < 2))

    @pl.loop(1, gg // 6 - 1)
    def _six(i):
        g0 = i * 6
        for u in range(6):
            step(g0 + u, u)

    for u in range(6):
        step(gg - 6 + u, u, no_next=(u >= 5), no_idx3=(u >= 3))

    s_wait(4, 1)
    s_wait(5, 2)

    plsc.subcore_barrier()

    pltpu.sync_copy(acc_sh.at[pl.ds(sid * rpt, rpt)],
                    out_hbm.at[cid, pl.ds(sid * rpt, rpt)])


def _make_compute_l1(bb):
  def _compute(rows_v, adv_v):
    @pl.loop(0, bb, unroll=4)
    def _edge(e):
        as16 = rows_v[e, pl.ds(IN, 16)]
        ad16 = adv_v[e, pl.ds(0, 16)]
        al = as16 + ad16
        al = jnp.where(al > 0, al, 0.2 * al)
        w16 = jnp.exp(al)
        rows_v[e, pl.ds(IN, 16)] = w16
        for h in range(HEADS):
            wv = w16[h]
            rows_v[e, pl.ds(h * 16, 16)] = rows_v[e, pl.ds(h * 16, 16)] * wv
  return _compute


def _make_compute_l2(bb):
  def _compute(rows_v, adv_v):
    lane = lax.iota(jnp.int32, 16)
    is8 = lane == 8

    @pl.loop(0, bb, unroll=4)
    def _edge(e):
        r2 = rows_v[e, pl.ds(32, 16)]
        ad16 = adv_v[e, pl.ds(0, 16)]
        s = r2 + ad16
        al = jnp.where(s > 0, s, 0.2 * s)
        w16 = jnp.exp(al)
        wv = w16[8]
        rows_v[e, pl.ds(0, 16)] = rows_v[e, pl.ds(0, 16)] * wv
        rows_v[e, pl.ds(16, 16)] = rows_v[e, pl.ds(16, 16)] * wv
        rows_v[e, pl.ds(32, 16)] = jnp.where(is8, w16, r2 * wv)
  return _compute


def _make_sc_kernel(dcols, bb, gg, compute_chunk, name):
    return pl.kernel(
        functools.partial(_sc_edge_body, dcols=dcols, bb=bb, gg=gg,
                          compute_chunk=compute_chunk),
        out_type=jax.ShapeDtypeStruct((NCORES, NPAD, dcols), jnp.float32),
        mesh=plsc.VectorSubcoreMesh(**_SC_MESH),
        scratch_types=[
            pltpu.VMEM((bb, dcols), jnp.float32),
            pltpu.VMEM((bb, dcols), jnp.float32),
            pltpu.VMEM((bb, dcols), jnp.float32),
            pltpu.VMEM((bb, DD), jnp.float32),
            pltpu.VMEM((bb, DD), jnp.float32),
            pltpu.VMEM((bb, DD), jnp.float32),
            pltpu.VMEM((6, bb), jnp.int32),
            pltpu.VMEM((6, bb), jnp.int32),
            pltpu.VMEM_SHARED((NPAD, dcols), jnp.float32),
            pltpu.SemaphoreType.DMA,
            pltpu.SemaphoreType.DMA,
            pltpu.SemaphoreType.DMA,
            pltpu.SemaphoreType.DMA,
            pltpu.SemaphoreType.DMA,
            pltpu.SemaphoreType.DMA,
            pltpu.SemaphoreType.DMA,
            pltpu.SemaphoreType.DMA,
            pltpu.SemaphoreType.DMA,
            pltpu.SemaphoreType.DMA,
            pltpu.SemaphoreType.DMA,
            pltpu.SemaphoreType.DMA,
        ],
        compiler_params=pltpu.CompilerParams(use_tc_tiling_on_sc=False),
        name=name,
    )


def kernel(x, edge_index, W1, att_src1, att_dst1, b1, W2, att_src2, att_dst2,
           b2):
    f32 = jnp.float32
    loop = jnp.arange(N, dtype=jnp.int32)
    padv = PADROW + jnp.arange(ET - E - N, dtype=jnp.int32) % NPADROWS
    src = jnp.concatenate([edge_index[0].astype(jnp.int32), loop, padv])
    dst = jnp.concatenate([edge_index[1].astype(jnp.int32), loop, padv])
    src1 = src.reshape(NTILES * G1, B1)
    dst1 = dst.reshape(NTILES * G1, B1)
    src2 = src.reshape(NTILES * G2, B2)
    dst2 = dst.reshape(NTILES * G2, B2)

    x_pad = jnp.zeros((NPAD, IN), f32).at[:N].set(x)

    eye8 = jnp.eye(HEADS, 16, dtype=f32)
    msrc = jnp.einsum("hd,hc->hdc", att_src1, eye8).reshape(IN, 16)
    mdst = jnp.einsum("hd,hc->hdc", att_dst1, eye8).reshape(IN, 16)
    rep = jnp.kron(jnp.eye(HEADS, dtype=f32), jnp.ones((1, HID), f32))
    p2 = jnp.concatenate(
        [jnp.eye(OUT, dtype=f32), att_src2.T, jnp.zeros((OUT, 7), f32)],
        axis=1)
    q2 = jnp.concatenate(
        [jnp.zeros((OUT, 8), f32), att_dst2.T, jnp.zeros((OUT, 7), f32)],
        axis=1)
    b1r = b1.reshape(1, IN)
    b2r = b2.reshape(1, OUT)

    RB = 640
    grid = NPAD // RB

    ts1, td1 = pl.pallas_call(
        _tca_body,
        grid=(grid,),
        in_specs=[
            pl.BlockSpec((RB, IN), lambda i: (i, 0)),
            pl.BlockSpec((IN, IN), lambda i: (0, 0)),
            pl.BlockSpec((IN, 16), lambda i: (0, 0)),
            pl.BlockSpec((IN, 16), lambda i: (0, 0)),
        ],
        out_specs=[
            pl.BlockSpec((RB, D1), lambda i: (i, 0)),
            pl.BlockSpec((RB, DD), lambda i: (i, 0)),
        ],
        out_shape=[
            jax.ShapeDtypeStruct((NPAD, D1), f32),
            jax.ShapeDtypeStruct((NPAD, DD), f32),
        ],
    )(x_pad, W1, msrc, mdst)

    acc1 = _make_sc_kernel(D1, B1, G1, _make_compute_l1(B1), "sc_gat_l1")(
        ts1, td1, src1, dst1)

    ts2, td2 = pl.pallas_call(
        _tcb_body,
        grid=(grid,),
        in_specs=[
            pl.BlockSpec((NCORES, RB, D1), lambda i: (0, i, 0)),
            pl.BlockSpec((HEADS, IN), lambda i: (0, 0)),
            pl.BlockSpec((1, IN), lambda i: (0, 0)),
            pl.BlockSpec((IN, OUT), lambda i: (0, 0)),
            pl.BlockSpec((OUT, D2), lambda i: (0, 0)),
            pl.BlockSpec((OUT, DD), lambda i: (0, 0)),
        ],
        out_specs=[
            pl.BlockSpec((RB, D2), lambda i: (i, 0)),
            pl.BlockSpec((RB, DD), lambda i: (i, 0)),
        ],
        out_shape=[
            jax.ShapeDtypeStruct((NPAD, D2), f32),
            jax.ShapeDtypeStruct((NPAD, DD), f32),
        ],
    )(acc1, rep, b1r, W2, p2, q2)

    acc2 = _make_sc_kernel(D2, B2, G2, _make_compute_l2(B2), "sc_gat_l2")(
        ts2, td2, src2, dst2)

    out = pl.pallas_call(
        _tcc_body,
        grid=(grid,),
        in_specs=[
            pl.BlockSpec((NCORES, RB, D2), lambda i: (0, i, 0)),
            pl.BlockSpec((1, OUT), lambda i: (0, 0)),
        ],
        out_specs=pl.BlockSpec((RB, OUT), lambda i: (i, 0)),
        out_shape=jax.ShapeDtypeStruct((NPAD, OUT), f32),
    )(acc2, b2r)

    return out[:N]
